# baseline (speedup 1.0000x reference)
import functools

import jax
import jax.numpy as jnp
from jax import lax
from jax.experimental import pallas as pl
from jax.experimental.pallas import tpu as pltpu

_BN_EPS = 1e-5
_INV_SQRT2 = 0.7071067811865476
_VMEM_LIMIT = 64 * 1024 * 1024
_ROW_TILE = 4096


def _round_up(a, b):
    return (a + b - 1) // b * b


def _gelu_exact(x):
    return 0.5 * x * (1.0 + lax.erf(x * _INV_SQRT2))


def _stats_kernel(x_ref, g_ref, bt_ref, scale_ref, shift_ref, s1_scr, s2_scr,
                  *, n_rows, row_tile, masked):
    j = pl.program_id(0)

    @pl.when(j == 0)
    def _():
        s1_scr[...] = jnp.zeros_like(s1_scr)
        s2_scr[...] = jnp.zeros_like(s2_scr)

    xt = x_ref[...]
    if masked:
        row_ids = lax.broadcasted_iota(jnp.int32, xt.shape, 0) + j * row_tile
        xt = jnp.where(row_ids < n_rows, xt, 0.0)

    x3 = xt.reshape(row_tile // 8, 8, -1)
    s1_scr[...] += jnp.sum(x3, axis=0)
    s2_scr[...] += jnp.sum(x3 * x3, axis=0)

    @pl.when(j == pl.num_programs(0) - 1)
    def _():
        inv_n = 1.0 / n_rows
        m = jnp.sum(s1_scr[...], axis=0, keepdims=True) * inv_n
        v = jnp.sum(s2_scr[...], axis=0, keepdims=True) * inv_n - m * m
        sc = g_ref[...] * lax.rsqrt(v + _BN_EPS)
        scale_ref[...] = sc
        shift_ref[...] = bt_ref[...] - m * sc


def _mlp_kernel(x_ref, scale_ref, shift_ref, w0_ref, b0_ref, w1_ref, b1_ref,
                out_ref, *, chunk):
    sc = scale_ref[...]
    sh = shift_ref[...]
    w0 = w0_ref[...]
    b0 = b0_ref[...]
    w1 = w1_ref[...]
    b1 = b1_ref[...]
    for k in range(x_ref.shape[0] // chunk):
        sl = pl.ds(k * chunk, chunk)
        h = x_ref[sl, :] * sc + sh
        h = jnp.dot(h.astype(jnp.bfloat16), w0,
                    preferred_element_type=jnp.float32) + b0
        h = _gelu_exact(h.astype(jnp.bfloat16))
        h = jnp.dot(h, w1, preferred_element_type=jnp.float32) + b1
        out_ref[sl, :] = _gelu_exact(h.astype(jnp.bfloat16)).astype(
            out_ref.dtype)


def kernel(x, bn_gamma, bn_beta, w0, b0, w1, b1):
    B, S, D_in = x.shape
    N = B * S
    D_out = w1.shape[1]

    row_tile = max(8, min(_ROW_TILE, _round_up(N, 8)))
    N_pad = _round_up(N, row_tile * 2)
    T = N_pad // row_tile
    masked = N_pad != N

    x2 = x.reshape(N, D_in)
    if masked:
        x2 = jnp.pad(x2, ((0, N_pad - N), (0, 0)))

    stats_tile = 2 * row_tile
    T_stats = N_pad // stats_tile

    const = pl.Buffered(buffer_count=1)
    cspec = pl.BlockSpec((1, D_in), lambda j: (0, 0), pipeline_mode=const)

    scale, shift = pl.pallas_call(
        functools.partial(_stats_kernel, n_rows=N, row_tile=stats_tile,
                          masked=masked),
        out_shape=(jax.ShapeDtypeStruct((1, D_in), jnp.float32),
                   jax.ShapeDtypeStruct((1, D_in), jnp.float32)),
        grid=(T_stats,),
        in_specs=[pl.BlockSpec((stats_tile, D_in), lambda j: (j, 0)),
                  cspec, cspec],
        out_specs=(pl.BlockSpec((1, D_in), lambda j: (0, 0)),
                   pl.BlockSpec((1, D_in), lambda j: (0, 0))),
        scratch_shapes=[pltpu.VMEM((8, D_in), jnp.float32),
                        pltpu.VMEM((8, D_in), jnp.float32)],
        compiler_params=pltpu.CompilerParams(
            dimension_semantics=("arbitrary",),
            vmem_limit_bytes=_VMEM_LIMIT),
    )(x2, bn_gamma.reshape(1, D_in).astype(jnp.float32),
      bn_beta.reshape(1, D_in).astype(jnp.float32))

    out = pl.pallas_call(
        functools.partial(_mlp_kernel, chunk=min(1024, row_tile)),
        out_shape=jax.ShapeDtypeStruct((N_pad, D_out), x.dtype),
        grid=(T,),
        in_specs=[pl.BlockSpec((row_tile, D_in), lambda i: (i, 0)),
                  pl.BlockSpec((1, D_in), lambda i: (0, 0),
                               pipeline_mode=const),
                  pl.BlockSpec((1, D_in), lambda i: (0, 0),
                               pipeline_mode=const),
                  pl.BlockSpec(w0.shape, lambda i: (0, 0),
                               pipeline_mode=const),
                  pl.BlockSpec((1, b0.shape[0]), lambda i: (0, 0),
                               pipeline_mode=const),
                  pl.BlockSpec(w1.shape, lambda i: (0, 0),
                               pipeline_mode=const),
                  pl.BlockSpec((1, b1.shape[0]), lambda i: (0, 0),
                               pipeline_mode=const)],
        out_specs=pl.BlockSpec((row_tile, D_out), lambda i: (i, 0)),
        compiler_params=pltpu.CompilerParams(
            dimension_semantics=("parallel",),
            vmem_limit_bytes=_VMEM_LIMIT),
    )(x2, scale, shift,
      w0.astype(jnp.bfloat16), b0.reshape(1, -1).astype(jnp.float32),
      w1.astype(jnp.bfloat16), b1.reshape(1, -1).astype(jnp.float32))

    return out[:N].reshape(B, S, D_out)


# speedup vs baseline: 1.0126x; 1.0126x over previous
import functools

import jax
import jax.numpy as jnp
from jax import lax
from jax.experimental import pallas as pl
from jax.experimental.pallas import tpu as pltpu

_BN_EPS = 1e-5
_INV_SQRT2 = 0.7071067811865476
_VMEM_LIMIT = 64 * 1024 * 1024
_ROW_TILE = 4096


def _round_up(a, b):
    return (a + b - 1) // b * b


def _gelu_exact(x):
    return 0.5 * x * (1.0 + lax.erf(x * _INV_SQRT2))


def _stats_kernel(x_ref, g_ref, bt_ref, scale_ref, shift_ref, s1_scr, s2_scr,
                  *, n_rows, row_tile, masked):
    j = pl.program_id(0)

    @pl.when(j == 0)
    def _():
        s1_scr[...] = jnp.zeros_like(s1_scr)
        s2_scr[...] = jnp.zeros_like(s2_scr)

    xt = x_ref[...]
    if masked:
        row_ids = lax.broadcasted_iota(jnp.int32, xt.shape, 0) + j * row_tile
        xt = jnp.where(row_ids < n_rows, xt, 0.0)

    x3 = xt.reshape(row_tile // 8, 8, -1)
    s1_scr[...] += jnp.sum(x3, axis=0)
    s2_scr[...] += jnp.sum(x3 * x3, axis=0)

    @pl.when(j == pl.num_programs(0) - 1)
    def _():
        inv_n = 1.0 / n_rows
        m = jnp.sum(s1_scr[...], axis=0, keepdims=True) * inv_n
        v = jnp.sum(s2_scr[...], axis=0, keepdims=True) * inv_n - m * m
        sc = g_ref[...] * lax.rsqrt(v + _BN_EPS)
        scale_ref[...] = sc
        shift_ref[...] = bt_ref[...] - m * sc


def _mlp_kernel(x_ref, scale_ref, shift_ref, w0_ref, b0_ref, w1_ref, b1_ref,
                out_ref, *, chunk):
    sc = scale_ref[...]
    sh = shift_ref[...]
    w0 = w0_ref[...]
    b0 = b0_ref[...]
    w1 = w1_ref[...]
    b1 = b1_ref[...]
    for k in range(x_ref.shape[0] // chunk):
        sl = pl.ds(k * chunk, chunk)
        h = x_ref[sl, :] * sc + sh
        h = jnp.dot(h.astype(jnp.bfloat16), w0,
                    preferred_element_type=jnp.float32) + b0
        h = _gelu_exact(h.astype(jnp.bfloat16))
        h = jnp.dot(h, w1, preferred_element_type=jnp.float32) + b1
        out_ref[sl, :] = _gelu_exact(h.astype(jnp.bfloat16)).astype(
            out_ref.dtype)


def kernel(x, bn_gamma, bn_beta, w0, b0, w1, b1):
    B, S, D_in = x.shape
    N = B * S
    D_out = w1.shape[1]

    row_tile = max(8, min(_ROW_TILE, _round_up(N, 8)))
    N_pad = _round_up(N, row_tile * 2)
    T = N_pad // row_tile
    masked = N_pad != N

    x2 = x.reshape(N, D_in)
    if masked:
        x2 = jnp.pad(x2, ((0, N_pad - N), (0, 0)))

    stats_tile = row_tile
    T_stats = N_pad // stats_tile

    const = pl.Buffered(buffer_count=1)
    cspec = pl.BlockSpec((1, D_in), lambda j: (0, 0), pipeline_mode=const)

    scale, shift = pl.pallas_call(
        functools.partial(_stats_kernel, n_rows=N, row_tile=stats_tile,
                          masked=masked),
        out_shape=(jax.ShapeDtypeStruct((1, D_in), jnp.float32),
                   jax.ShapeDtypeStruct((1, D_in), jnp.float32)),
        grid=(T_stats,),
        in_specs=[pl.BlockSpec((stats_tile, D_in), lambda j: (j, 0)),
                  cspec, cspec],
        out_specs=(pl.BlockSpec((1, D_in), lambda j: (0, 0)),
                   pl.BlockSpec((1, D_in), lambda j: (0, 0))),
        scratch_shapes=[pltpu.VMEM((8, D_in), jnp.float32),
                        pltpu.VMEM((8, D_in), jnp.float32)],
        compiler_params=pltpu.CompilerParams(
            dimension_semantics=("arbitrary",),
            vmem_limit_bytes=_VMEM_LIMIT),
    )(x2, bn_gamma.reshape(1, D_in).astype(jnp.float32),
      bn_beta.reshape(1, D_in).astype(jnp.float32))

    out = pl.pallas_call(
        functools.partial(_mlp_kernel, chunk=min(512, row_tile)),
        out_shape=jax.ShapeDtypeStruct((N_pad, D_out), x.dtype),
        grid=(T,),
        in_specs=[pl.BlockSpec((row_tile, D_in), lambda i: (i, 0)),
                  pl.BlockSpec((1, D_in), lambda i: (0, 0),
                               pipeline_mode=const),
                  pl.BlockSpec((1, D_in), lambda i: (0, 0),
                               pipeline_mode=const),
                  pl.BlockSpec(w0.shape, lambda i: (0, 0),
                               pipeline_mode=const),
                  pl.BlockSpec((1, b0.shape[0]), lambda i: (0, 0),
                               pipeline_mode=const),
                  pl.BlockSpec(w1.shape, lambda i: (0, 0),
                               pipeline_mode=const),
                  pl.BlockSpec((1, b1.shape[0]), lambda i: (0, 0),
                               pipeline_mode=const)],
        out_specs=pl.BlockSpec((row_tile, D_out), lambda i: (i, 0)),
        compiler_params=pltpu.CompilerParams(
            dimension_semantics=("arbitrary",),
            vmem_limit_bytes=_VMEM_LIMIT),
    )(x2, scale, shift,
      w0.astype(jnp.bfloat16), b0.reshape(1, -1).astype(jnp.float32),
      w1.astype(jnp.bfloat16), b1.reshape(1, -1).astype(jnp.float32))

    return out[:N].reshape(B, S, D_out)


# speedup vs baseline: 1.0560x; 1.0429x over previous
import functools

import jax
import jax.numpy as jnp
from jax import lax
from jax.experimental import pallas as pl
from jax.experimental.pallas import tpu as pltpu

_BN_EPS = 1e-5
_INV_SQRT2 = 0.7071067811865476
_VMEM_LIMIT = 64 * 1024 * 1024
_ROW_TILE = 4096


def _round_up(a, b):
    return (a + b - 1) // b * b


def _gelu_exact(x):
    return 0.5 * x * (1.0 + lax.erf(x * _INV_SQRT2))


def _stats_kernel(x_ref, g_ref, bt_ref, w0_ref, b0_ref, w0b_ref, b0f_ref,
                  s1_scr, s2_scr, *, n_rows, row_tile, masked):
    j = pl.program_id(0)

    @pl.when(j == 0)
    def _():
        s1_scr[...] = jnp.zeros_like(s1_scr)
        s2_scr[...] = jnp.zeros_like(s2_scr)

    xt = x_ref[...]
    if masked:
        row_ids = lax.broadcasted_iota(jnp.int32, xt.shape, 0) + j * row_tile
        xt = jnp.where(row_ids < n_rows, xt, 0.0)

    x3 = xt.reshape(row_tile // 8, 8, -1)
    s1_scr[...] += jnp.sum(x3, axis=0)
    s2_scr[...] += jnp.sum(x3 * x3, axis=0)

    @pl.when(j == pl.num_programs(0) - 1)
    def _():
        inv_n = 1.0 / n_rows
        m = jnp.sum(s1_scr[...], axis=0, keepdims=True) * inv_n
        v = jnp.sum(s2_scr[...], axis=0, keepdims=True) * inv_n - m * m
        sc = g_ref[...] * lax.rsqrt(v + _BN_EPS)
        sh = bt_ref[...] - m * sc
        w0 = w0_ref[...]
        w0b_ref[...] = (w0 * jnp.transpose(sc)).astype(jnp.bfloat16)
        b0f_ref[...] = b0_ref[...] + jnp.dot(
            sh, w0, preferred_element_type=jnp.float32)


def _mlp_kernel(x_ref, w0_ref, b0_ref, w1_ref, b1_ref, out_ref, *, chunk):
    w0 = w0_ref[...]
    b0 = b0_ref[...]
    w1 = w1_ref[...]
    b1 = b1_ref[...]
    for k in range(x_ref.shape[0] // chunk):
        sl = pl.ds(k * chunk, chunk)
        h = jnp.dot(x_ref[sl, :].astype(jnp.bfloat16), w0,
                    preferred_element_type=jnp.float32) + b0
        h = _gelu_exact(h.astype(jnp.bfloat16))
        h = jnp.dot(h, w1, preferred_element_type=jnp.float32) + b1
        out_ref[sl, :] = _gelu_exact(h.astype(jnp.bfloat16)).astype(
            out_ref.dtype)


def kernel(x, bn_gamma, bn_beta, w0, b0, w1, b1):
    B, S, D_in = x.shape
    N = B * S
    D_out = w1.shape[1]

    row_tile = max(8, min(_ROW_TILE, _round_up(N, 8)))
    N_pad = _round_up(N, row_tile * 2)
    T = N_pad // row_tile
    masked = N_pad != N

    x2 = x.reshape(N, D_in)
    if masked:
        x2 = jnp.pad(x2, ((0, N_pad - N), (0, 0)))

    stats_tile = row_tile
    T_stats = N_pad // stats_tile

    const = pl.Buffered(buffer_count=1)
    cspec = pl.BlockSpec((1, D_in), lambda j: (0, 0), pipeline_mode=const)

    d1 = w0.shape[1]
    w0b, b0f = pl.pallas_call(
        functools.partial(_stats_kernel, n_rows=N, row_tile=stats_tile,
                          masked=masked),
        out_shape=(jax.ShapeDtypeStruct((D_in, d1), jnp.bfloat16),
                   jax.ShapeDtypeStruct((1, d1), jnp.float32)),
        grid=(T_stats,),
        in_specs=[pl.BlockSpec((stats_tile, D_in), lambda j: (j, 0)),
                  cspec, cspec,
                  pl.BlockSpec(w0.shape, lambda j: (0, 0),
                               pipeline_mode=const),
                  pl.BlockSpec((1, d1), lambda j: (0, 0),
                               pipeline_mode=const)],
        out_specs=(pl.BlockSpec((D_in, d1), lambda j: (0, 0)),
                   pl.BlockSpec((1, d1), lambda j: (0, 0))),
        scratch_shapes=[pltpu.VMEM((8, D_in), jnp.float32),
                        pltpu.VMEM((8, D_in), jnp.float32)],
        compiler_params=pltpu.CompilerParams(
            dimension_semantics=("arbitrary",),
            vmem_limit_bytes=_VMEM_LIMIT),
    )(x2, bn_gamma.reshape(1, D_in).astype(jnp.float32),
      bn_beta.reshape(1, D_in).astype(jnp.float32),
      w0.astype(jnp.float32), b0.reshape(1, d1).astype(jnp.float32))

    out = pl.pallas_call(
        functools.partial(_mlp_kernel, chunk=min(512, row_tile)),
        out_shape=jax.ShapeDtypeStruct((N_pad, D_out), x.dtype),
        grid=(T,),
        in_specs=[pl.BlockSpec((row_tile, D_in), lambda i: (i, 0)),
                  pl.BlockSpec(w0.shape, lambda i: (0, 0),
                               pipeline_mode=const),
                  pl.BlockSpec((1, d1), lambda i: (0, 0),
                               pipeline_mode=const),
                  pl.BlockSpec(w1.shape, lambda i: (0, 0),
                               pipeline_mode=const),
                  pl.BlockSpec((1, b1.shape[0]), lambda i: (0, 0),
                               pipeline_mode=const)],
        out_specs=pl.BlockSpec((row_tile, D_out), lambda i: (i, 0)),
        compiler_params=pltpu.CompilerParams(
            dimension_semantics=("arbitrary",),
            vmem_limit_bytes=_VMEM_LIMIT),
    )(x2, w0b, b0f,
      w1.astype(jnp.bfloat16), b1.reshape(1, -1).astype(jnp.float32))

    return out[:N].reshape(B, S, D_out)
